# Initial kernel scaffold; baseline (speedup 1.0000x reference)
import jax
import jax.numpy as jnp
from jax.experimental import pallas as pl


def kernel(word_ids, feat_ids, word_table, feat_table):
    raise NotImplementedError("write your pallas kernel here")


# speedup vs baseline: 2.0240x; 2.0240x over previous
import functools

import jax
import jax.numpy as jnp
from jax import lax
from jax.experimental import pallas as pl
from jax.experimental.pallas import tpu as pltpu
from jax.experimental.pallas import tpu_sc as plsc

WORD_DIM = 64
FEAT_DIM = 32
OUT_DIM = WORD_DIM + FEAT_DIM

NC = 2
NS = 16
NW = NC * NS

STEP = 128


@functools.partial(jax.jit, static_argnums=(4, 5))
def _lookup_concat(word_ids, feat_ids, word_table, feat_table, tpw, nstep):
    n_tokens = NW * tpw
    mesh = plsc.VectorSubcoreMesh(
        core_axis_name="c", subcore_axis_name="s", num_cores=NC, num_subcores=NS)

    @functools.partial(
        pl.kernel,
        out_type=jax.ShapeDtypeStruct((n_tokens, OUT_DIM), jnp.float32),
        mesh=mesh,
        compiler_params=pltpu.CompilerParams(use_tc_tiling_on_sc=False),
        scratch_types=[
            pltpu.VMEM((nstep, STEP), jnp.int32),
            pltpu.VMEM((nstep, STEP), jnp.int32),
            pltpu.VMEM((STEP, WORD_DIM), jnp.float32),
            pltpu.VMEM((STEP, FEAT_DIM), jnp.float32),
            pltpu.SemaphoreType.DMA,
            pltpu.SemaphoreType.DMA,
        ],
    )
    def k(word_ids_h, feat_ids_h, word_table_h, feat_table_h, out_h,
          widx, fidx, wbuf, fbuf, gsem, wsem):
        wid = lax.axis_index("s") * NC + lax.axis_index("c")
        base = wid * tpw
        pltpu.sync_copy(word_ids_h.at[wid], widx)
        pltpu.sync_copy(feat_ids_h.at[wid], fidx)

        def body(i, carry):
            g1 = pltpu.async_copy(word_table_h.at[widx.at[i]], wbuf, gsem)
            g2 = pltpu.async_copy(feat_table_h.at[fidx.at[i]], fbuf, gsem)
            g1.wait()
            g2.wait()
            ob = base + i * STEP
            w1 = pltpu.async_copy(
                wbuf, out_h.at[pl.ds(ob, STEP), pl.ds(0, WORD_DIM)], wsem)
            w2 = pltpu.async_copy(
                fbuf, out_h.at[pl.ds(ob, STEP), pl.ds(WORD_DIM, FEAT_DIM)], wsem)
            w1.wait()
            w2.wait()
            return carry

        lax.fori_loop(0, nstep, body, 0)

    return k(word_ids, feat_ids, word_table, feat_table)


def kernel(word_ids, feat_ids, word_table, feat_table):
    b, s = word_ids.shape
    n = b * s
    chunk = NW * STEP
    n_pad = ((n + chunk - 1) // chunk) * chunk
    wids = word_ids.reshape(-1).astype(jnp.int32)
    fids = feat_ids.reshape(-1).astype(jnp.int32)
    if n_pad != n:
        wids = jnp.pad(wids, (0, n_pad - n))
        fids = jnp.pad(fids, (0, n_pad - n))
    tpw = n_pad // NW
    nstep = tpw // STEP
    out = _lookup_concat(
        wids.reshape(NW, nstep, STEP),
        fids.reshape(NW, nstep, STEP),
        word_table, feat_table, tpw, nstep)
    return out[:n].reshape(b, s, OUT_DIM)


# speedup vs baseline: 2.1246x; 1.0497x over previous
import functools

import jax
import jax.numpy as jnp
from jax import lax
from jax.experimental import pallas as pl
from jax.experimental.pallas import tpu as pltpu
from jax.experimental.pallas import tpu_sc as plsc

WORD_DIM = 64
FEAT_DIM = 32
OUT_DIM = WORD_DIM + FEAT_DIM

NC = 2
NS = 16
NW = NC * NS

STEP = 128
K = 4


def _mesh():
    return plsc.VectorSubcoreMesh(
        core_axis_name="c", subcore_axis_name="s", num_cores=NC, num_subcores=NS)


@functools.partial(jax.jit, static_argnums=(4, 5))
def _lookup_concat_pipelined(word_ids, feat_ids, word_table, feat_table,
                             tpw, nstep):
    n_tokens = NW * tpw
    ngrp = nstep // K

    @functools.partial(
        pl.kernel,
        out_type=jax.ShapeDtypeStruct((n_tokens, OUT_DIM), jnp.float32),
        mesh=_mesh(),
        compiler_params=pltpu.CompilerParams(use_tc_tiling_on_sc=False),
        scratch_types=[
            pltpu.VMEM((nstep, STEP), jnp.int32),
            pltpu.VMEM((nstep, STEP), jnp.int32),
            pltpu.VMEM((2, K, STEP, WORD_DIM), jnp.float32),
            pltpu.VMEM((2, K, STEP, FEAT_DIM), jnp.float32),
            pltpu.SemaphoreType.DMA,
            pltpu.SemaphoreType.DMA,
            pltpu.SemaphoreType.DMA,
            pltpu.SemaphoreType.DMA,
        ],
    )
    def k(word_ids_h, feat_ids_h, word_table_h, feat_table_h, out_h,
          widx, fidx, wbuf, fbuf, gsem0, gsem1, wsem0, wsem1):
        wid = lax.axis_index("s") * NC + lax.axis_index("c")
        base = wid * tpw
        pltpu.sync_copy(word_ids_h.at[wid], widx)
        pltpu.sync_copy(feat_ids_h.at[wid], fidx)
        gsems = (gsem0, gsem1)
        wsems = (wsem0, wsem1)

        def gather_descs(g, s, b):
            i = g * K + b
            return (
                pltpu.make_async_copy(
                    word_table_h.at[widx.at[i]], wbuf.at[s, b], gsems[s]),
                pltpu.make_async_copy(
                    feat_table_h.at[fidx.at[i]], fbuf.at[s, b], gsems[s]),
            )

        def write_descs(g, s, b):
            ob = base + (g * K + b) * STEP
            return (
                pltpu.make_async_copy(
                    wbuf.at[s, b],
                    out_h.at[pl.ds(ob, STEP), pl.ds(0, WORD_DIM)], wsems[s]),
                pltpu.make_async_copy(
                    fbuf.at[s, b],
                    out_h.at[pl.ds(ob, STEP), pl.ds(WORD_DIM, FEAT_DIM)],
                    wsems[s]),
            )

        def fire_gathers(g, s):
            for b in range(K):
                for d in gather_descs(g, s, b):
                    d.start()

        def drain_gathers(g, s):
            for b in range(K):
                for d in gather_descs(g, s, b):
                    d.wait()

        def fire_writes(g, s):
            for b in range(K):
                for d in write_descs(g, s, b):
                    d.start()

        def drain_writes(g, s):
            for b in range(K):
                for d in write_descs(g, s, b):
                    d.wait()

        fire_gathers(0, 0)
        drain_gathers(0, 0)
        fire_gathers(1, 1)
        fire_writes(0, 0)
        drain_gathers(1, 1)
        drain_writes(0, 0)
        fire_gathers(2, 0)
        fire_writes(1, 1)

        def body(t, carry):
            g = 2 * t + 2
            drain_gathers(g, 0)
            drain_writes(g - 1, 1)
            fire_gathers(g + 1, 1)
            fire_writes(g, 0)
            drain_gathers(g + 1, 1)
            drain_writes(g, 0)
            fire_gathers(g + 2, 0)
            fire_writes(g + 1, 1)
            return carry

        lax.fori_loop(0, (ngrp - 4) // 2, body, 0)

        g = ngrp - 2
        drain_gathers(g, 0)
        drain_writes(g - 1, 1)
        fire_gathers(g + 1, 1)
        fire_writes(g, 0)
        drain_gathers(g + 1, 1)
        drain_writes(g, 0)
        fire_writes(g + 1, 1)
        drain_writes(g + 1, 1)

    return k(word_ids, feat_ids, word_table, feat_table)


@functools.partial(jax.jit, static_argnums=(4, 5))
def _lookup_concat_simple(word_ids, feat_ids, word_table, feat_table,
                          tpw, nstep):
    n_tokens = NW * tpw

    @functools.partial(
        pl.kernel,
        out_type=jax.ShapeDtypeStruct((n_tokens, OUT_DIM), jnp.float32),
        mesh=_mesh(),
        compiler_params=pltpu.CompilerParams(use_tc_tiling_on_sc=False),
        scratch_types=[
            pltpu.VMEM((nstep, STEP), jnp.int32),
            pltpu.VMEM((nstep, STEP), jnp.int32),
            pltpu.VMEM((STEP, WORD_DIM), jnp.float32),
            pltpu.VMEM((STEP, FEAT_DIM), jnp.float32),
            pltpu.SemaphoreType.DMA,
            pltpu.SemaphoreType.DMA,
        ],
    )
    def k(word_ids_h, feat_ids_h, word_table_h, feat_table_h, out_h,
          widx, fidx, wbuf, fbuf, gsem, wsem):
        wid = lax.axis_index("s") * NC + lax.axis_index("c")
        base = wid * tpw
        pltpu.sync_copy(word_ids_h.at[wid], widx)
        pltpu.sync_copy(feat_ids_h.at[wid], fidx)

        def body(i, carry):
            g1 = pltpu.async_copy(word_table_h.at[widx.at[i]], wbuf, gsem)
            g2 = pltpu.async_copy(feat_table_h.at[fidx.at[i]], fbuf, gsem)
            g1.wait()
            g2.wait()
            ob = base + i * STEP
            w1 = pltpu.async_copy(
                wbuf, out_h.at[pl.ds(ob, STEP), pl.ds(0, WORD_DIM)], wsem)
            w2 = pltpu.async_copy(
                fbuf, out_h.at[pl.ds(ob, STEP), pl.ds(WORD_DIM, FEAT_DIM)], wsem)
            w1.wait()
            w2.wait()
            return carry

        lax.fori_loop(0, nstep, body, 0)

    return k(word_ids, feat_ids, word_table, feat_table)


def kernel(word_ids, feat_ids, word_table, feat_table):
    b, s = word_ids.shape
    n = b * s
    chunk = NW * STEP
    n_pad = ((n + chunk - 1) // chunk) * chunk
    wids = word_ids.reshape(-1).astype(jnp.int32)
    fids = feat_ids.reshape(-1).astype(jnp.int32)
    if n_pad != n:
        wids = jnp.pad(wids, (0, n_pad - n))
        fids = jnp.pad(fids, (0, n_pad - n))
    tpw = n_pad // NW
    nstep = tpw // STEP
    if nstep % K == 0 and nstep // K >= 4:
        fn = _lookup_concat_pipelined
    else:
        fn = _lookup_concat_simple
    out = fn(
        wids.reshape(NW, nstep, STEP),
        fids.reshape(NW, nstep, STEP),
        word_table, feat_table, tpw, nstep)
    return out[:n].reshape(b, s, OUT_DIM)


# speedup vs baseline: 2.1280x; 1.0016x over previous
import functools

import jax
import jax.numpy as jnp
from jax import lax
from jax.experimental import pallas as pl
from jax.experimental.pallas import tpu as pltpu
from jax.experimental.pallas import tpu_sc as plsc

WORD_DIM = 64
FEAT_DIM = 32
OUT_DIM = WORD_DIM + FEAT_DIM

NC = 2
NS = 16
NW = NC * NS

STEP = 512
K = 1


def _mesh():
    return plsc.VectorSubcoreMesh(
        core_axis_name="c", subcore_axis_name="s", num_cores=NC, num_subcores=NS)


@functools.partial(jax.jit, static_argnums=(4, 5))
def _lookup_concat_pipelined(word_ids, feat_ids, word_table, feat_table,
                             tpw, nstep):
    n_tokens = NW * tpw
    ngrp = nstep // K

    @functools.partial(
        pl.kernel,
        out_type=jax.ShapeDtypeStruct((n_tokens, OUT_DIM), jnp.float32),
        mesh=_mesh(),
        compiler_params=pltpu.CompilerParams(use_tc_tiling_on_sc=False),
        scratch_types=[
            pltpu.VMEM((nstep, STEP), jnp.int32),
            pltpu.VMEM((nstep, STEP), jnp.int32),
            pltpu.VMEM((2, K, STEP, WORD_DIM), jnp.float32),
            pltpu.VMEM((2, K, STEP, FEAT_DIM), jnp.float32),
            pltpu.SemaphoreType.DMA,
            pltpu.SemaphoreType.DMA,
            pltpu.SemaphoreType.DMA,
            pltpu.SemaphoreType.DMA,
        ],
    )
    def k(word_ids_h, feat_ids_h, word_table_h, feat_table_h, out_h,
          widx, fidx, wbuf, fbuf, gsem0, gsem1, wsem0, wsem1):
        wid = lax.axis_index("s") * NC + lax.axis_index("c")
        base = wid * tpw
        pltpu.sync_copy(word_ids_h.at[wid], widx)
        pltpu.sync_copy(feat_ids_h.at[wid], fidx)
        gsems = (gsem0, gsem1)
        wsems = (wsem0, wsem1)

        def gather_descs(g, s, b):
            i = g * K + b
            return (
                pltpu.make_async_copy(
                    word_table_h.at[widx.at[i]], wbuf.at[s, b], gsems[s]),
                pltpu.make_async_copy(
                    feat_table_h.at[fidx.at[i]], fbuf.at[s, b], gsems[s]),
            )

        def write_descs(g, s, b):
            ob = base + (g * K + b) * STEP
            return (
                pltpu.make_async_copy(
                    wbuf.at[s, b],
                    out_h.at[pl.ds(ob, STEP), pl.ds(0, WORD_DIM)], wsems[s]),
                pltpu.make_async_copy(
                    fbuf.at[s, b],
                    out_h.at[pl.ds(ob, STEP), pl.ds(WORD_DIM, FEAT_DIM)],
                    wsems[s]),
            )

        def fire_gathers(g, s):
            for b in range(K):
                for d in gather_descs(g, s, b):
                    d.start()

        def drain_gathers(g, s):
            for b in range(K):
                for d in gather_descs(g, s, b):
                    d.wait()

        def fire_writes(g, s):
            for b in range(K):
                for d in write_descs(g, s, b):
                    d.start()

        def drain_writes(g, s):
            for b in range(K):
                for d in write_descs(g, s, b):
                    d.wait()

        fire_gathers(0, 0)
        drain_gathers(0, 0)
        fire_gathers(1, 1)
        fire_writes(0, 0)
        drain_gathers(1, 1)
        drain_writes(0, 0)
        fire_gathers(2, 0)
        fire_writes(1, 1)

        def body(t, carry):
            g = 2 * t + 2
            drain_gathers(g, 0)
            drain_writes(g - 1, 1)
            fire_gathers(g + 1, 1)
            fire_writes(g, 0)
            drain_gathers(g + 1, 1)
            drain_writes(g, 0)
            fire_gathers(g + 2, 0)
            fire_writes(g + 1, 1)
            return carry

        lax.fori_loop(0, (ngrp - 4) // 2, body, 0)

        g = ngrp - 2
        drain_gathers(g, 0)
        drain_writes(g - 1, 1)
        fire_gathers(g + 1, 1)
        fire_writes(g, 0)
        drain_gathers(g + 1, 1)
        drain_writes(g, 0)
        fire_writes(g + 1, 1)
        drain_writes(g + 1, 1)

    return k(word_ids, feat_ids, word_table, feat_table)


@functools.partial(jax.jit, static_argnums=(4, 5))
def _lookup_concat_simple(word_ids, feat_ids, word_table, feat_table,
                          tpw, nstep):
    n_tokens = NW * tpw

    @functools.partial(
        pl.kernel,
        out_type=jax.ShapeDtypeStruct((n_tokens, OUT_DIM), jnp.float32),
        mesh=_mesh(),
        compiler_params=pltpu.CompilerParams(use_tc_tiling_on_sc=False),
        scratch_types=[
            pltpu.VMEM((nstep, STEP), jnp.int32),
            pltpu.VMEM((nstep, STEP), jnp.int32),
            pltpu.VMEM((STEP, WORD_DIM), jnp.float32),
            pltpu.VMEM((STEP, FEAT_DIM), jnp.float32),
            pltpu.SemaphoreType.DMA,
            pltpu.SemaphoreType.DMA,
        ],
    )
    def k(word_ids_h, feat_ids_h, word_table_h, feat_table_h, out_h,
          widx, fidx, wbuf, fbuf, gsem, wsem):
        wid = lax.axis_index("s") * NC + lax.axis_index("c")
        base = wid * tpw
        pltpu.sync_copy(word_ids_h.at[wid], widx)
        pltpu.sync_copy(feat_ids_h.at[wid], fidx)

        def body(i, carry):
            g1 = pltpu.async_copy(word_table_h.at[widx.at[i]], wbuf, gsem)
            g2 = pltpu.async_copy(feat_table_h.at[fidx.at[i]], fbuf, gsem)
            g1.wait()
            g2.wait()
            ob = base + i * STEP
            w1 = pltpu.async_copy(
                wbuf, out_h.at[pl.ds(ob, STEP), pl.ds(0, WORD_DIM)], wsem)
            w2 = pltpu.async_copy(
                fbuf, out_h.at[pl.ds(ob, STEP), pl.ds(WORD_DIM, FEAT_DIM)], wsem)
            w1.wait()
            w2.wait()
            return carry

        lax.fori_loop(0, nstep, body, 0)

    return k(word_ids, feat_ids, word_table, feat_table)


def kernel(word_ids, feat_ids, word_table, feat_table):
    b, s = word_ids.shape
    n = b * s
    chunk = NW * STEP
    n_pad = ((n + chunk - 1) // chunk) * chunk
    wids = word_ids.reshape(-1).astype(jnp.int32)
    fids = feat_ids.reshape(-1).astype(jnp.int32)
    if n_pad != n:
        wids = jnp.pad(wids, (0, n_pad - n))
        fids = jnp.pad(fids, (0, n_pad - n))
    tpw = n_pad // NW
    nstep = tpw // STEP
    if nstep % K == 0 and nstep // K >= 4:
        fn = _lookup_concat_pipelined
    else:
        fn = _lookup_concat_simple
    out = fn(
        wids.reshape(NW, nstep, STEP),
        fids.reshape(NW, nstep, STEP),
        word_table, feat_table, tpw, nstep)
    return out[:n].reshape(b, s, OUT_DIM)
